# baseline (speedup 1.0000x reference)
import jax
import jax.numpy as jnp
from jax import lax
from jax.experimental import pallas as pl
from jax.experimental.pallas import tpu as pltpu
from jax.experimental.pallas import tpu_sc as plsc

B, N, D = 8, 2048, 1024
NBOX = 6
THRESH = 0.5
L = 16
NCHUNK = N // L

_SUPP_OFF = 16.0
_PICKED = -1e30
_LIVE_MIN = -1.0


def _sc_nms_gather_body(scores_hbm, boxes_hbm, feat_hbm,
                        fsel_hbm, bsel_hbm,
                        orig_v, s_v, x1_v, y1_v, x2_v, y2_v, area_v,
                        boxes_v, bsel_v, idx_v, frows_v, sem):
    wid = lax.axis_index("s")
    lane = lax.iota(jnp.int32, L)

    @pl.when(wid < B)
    def _():
        b = wid
        pltpu.sync_copy(scores_hbm.at[b], orig_v)
        pltpu.sync_copy(scores_hbm.at[b], s_v)
        pltpu.sync_copy(boxes_hbm.at[b], boxes_v)

        neg_inf = jnp.full((L,), -jnp.inf, jnp.float32)
        zeros_i = jnp.zeros((L,), jnp.int32)

        def bev_body(c, carry):
            bv, bi = carry
            rows = c * L + lane
            base = rows * 7
            xc = plsc.load_gather(boxes_v, [base])
            zc = plsc.load_gather(boxes_v, [base + 2])
            ww = plsc.load_gather(boxes_v, [base + 4])
            ll = plsc.load_gather(boxes_v, [base + 5])
            x1 = xc - ll / 2.0
            y1 = zc - ww / 2.0
            x2 = xc + ll / 2.0
            y2 = zc + ww / 2.0
            sl = pl.ds(c * L, L)
            x1_v[sl] = x1
            y1_v[sl] = y1
            x2_v[sl] = x2
            y2_v[sl] = y2
            area_v[sl] = (x2 - x1) * (y2 - y1)
            v = s_v[sl]
            gt = v > bv
            return (jnp.where(gt, v, bv), jnp.where(gt, rows, bi))

        bv, bi = lax.fori_loop(0, NCHUNK, bev_body, (neg_inf, zeros_i))

        picks = zeros_i
        for r in range(NBOX):
            m = jnp.max(bv)
            i = jnp.min(jnp.where(bv == m, bi, N))
            picks = jnp.where(lane == r, i, picks)

            if r < NBOX - 1:
                iv = jnp.full((L,), i, jnp.int32)
                px1 = plsc.load_gather(x1_v, [iv])
                py1 = plsc.load_gather(y1_v, [iv])
                px2 = plsc.load_gather(x2_v, [iv])
                py2 = plsc.load_gather(y2_v, [iv])
                parea = plsc.load_gather(area_v, [iv])
                porig = plsc.load_gather(orig_v, [iv])
                live = m > _LIVE_MIN
                plsc.store_scatter(s_v, [iv], jnp.full((L,), _PICKED, jnp.float32),
                                   mask=lane == 0)

                def fused_body(c, carry):
                    bv, bi = carry
                    sl = pl.ds(c * L, L)
                    jx1 = x1_v[sl]
                    jy1 = y1_v[sl]
                    jx2 = x2_v[sl]
                    jy2 = y2_v[sl]
                    jarea = area_v[sl]
                    jorig = orig_v[sl]
                    js = s_v[sl]
                    inter = (jnp.maximum(jnp.minimum(px2, jx2) - jnp.maximum(px1, jx1), 0.0)
                             * jnp.maximum(jnp.minimum(py2, jy2) - jnp.maximum(py1, jy1), 0.0))
                    iou = inter / (parea + jarea - inter + 1e-8)
                    cond = (iou > THRESH) & (jorig < porig) & (js > _LIVE_MIN) & live
                    js = jnp.where(cond, js - _SUPP_OFF, js)
                    s_v[sl] = js
                    gt = js > bv
                    idx = c * L + lane
                    return (jnp.where(gt, js, bv), jnp.where(gt, idx, bi))

                bv, bi = lax.fori_loop(0, NCHUNK, fused_body, (neg_inf, zeros_i))

        for c in range(7):
            vals = plsc.load_gather(boxes_v, [picks * 7 + c])
            plsc.store_scatter(bsel_v, [lane * 7 + c], vals, mask=lane < NBOX)
        pltpu.sync_copy(bsel_v, bsel_hbm.at[b])

        gidx = picks + b * N
        plsc.store_scatter(idx_v, [lane], gidx, mask=lane < 8)
        pltpu.async_copy(feat_hbm.at[idx_v], frows_v, sem).wait()
        pltpu.sync_copy(frows_v.at[pl.ds(0, NBOX)], fsel_hbm.at[b])


@jax.jit
def _sc_nms_gather(scores, boxes3d, feat_flat):
    mesh = plsc.VectorSubcoreMesh(core_axis_name="c", subcore_axis_name="s",
                                  num_cores=1)
    return pl.kernel(
        _sc_nms_gather_body,
        out_type=[
            jax.ShapeDtypeStruct((B, NBOX, D), jnp.float32),
            jax.ShapeDtypeStruct((B, 48), jnp.float32),
        ],
        mesh=mesh,
        compiler_params=pltpu.CompilerParams(needs_layout_passes=False),
        scratch_types=[
            pltpu.VMEM((N,), jnp.float32),
            pltpu.VMEM((N,), jnp.float32),
            pltpu.VMEM((N,), jnp.float32),
            pltpu.VMEM((N,), jnp.float32),
            pltpu.VMEM((N,), jnp.float32),
            pltpu.VMEM((N,), jnp.float32),
            pltpu.VMEM((N,), jnp.float32),
            pltpu.VMEM((N * 7,), jnp.float32),
            pltpu.VMEM((48,), jnp.float32),
            pltpu.VMEM((8,), jnp.int32),
            pltpu.VMEM((8, D), jnp.float32),
            pltpu.SemaphoreType.DMA,
        ],
    )(scores, boxes3d, feat_flat)


def _tc_mlp_body(h_ref, w1_ref, b1_ref, g1_ref, be1_ref,
                 w2_ref, b2_ref, g2_ref, be2_ref,
                 w3_ref, b3_ref, w4_ref, b4_ref, x_ref, conf_ref):
    h = h_ref[...]

    def bn(x, g, b):
        mu = jnp.mean(x, axis=0)
        var = jnp.mean((x - mu[None, :]) ** 2, axis=0)
        return (x - mu[None, :]) / jnp.sqrt(var + 1e-5) * g[None, :] + b[None, :]

    h1 = jnp.dot(h, w1_ref[...], preferred_element_type=jnp.float32) + b1_ref[...][None, :]
    h1 = jnp.maximum(bn(h1, g1_ref[...], be1_ref[...]), 0.0)
    h2 = jnp.dot(h1, w2_ref[...], preferred_element_type=jnp.float32) + b2_ref[...][None, :]
    h2 = jnp.maximum(bn(h2, g2_ref[...], be2_ref[...]), 0.0)
    x_ref[...] = (jnp.dot(h2, w3_ref[...], preferred_element_type=jnp.float32)
                  + b3_ref[...][None, :])
    logits = (jnp.dot(h2, w4_ref[...], preferred_element_type=jnp.float32)
              + b4_ref[...][None, :])
    mx = jnp.max(logits, axis=1, keepdims=True)
    e = jnp.exp(logits - mx)
    conf_ref[...] = e / jnp.sum(e, axis=1, keepdims=True)


@jax.jit
def _tc_mlp(h, W1, b1, g1, be1, W2, b2, g2, be2, W3, b3, W4, b4):
    return pl.pallas_call(
        _tc_mlp_body,
        out_shape=[
            jax.ShapeDtypeStruct((B * NBOX, 7), jnp.float32),
            jax.ShapeDtypeStruct((B * NBOX, 2), jnp.float32),
        ],
    )(h, W1, b1, g1, be1, W2, b2, g2, be2, W3, b3, W4, b4)


def kernel(features, boxes3d, scores, W1, b1, g1, be1, W2, b2, g2, be2, W3, b3, W4, b4):
    feat_flat = features.reshape(B * N, D)
    fsel, bsel_pad = _sc_nms_gather(scores, boxes3d.reshape(B, N * 7), feat_flat)
    bsel = bsel_pad[:, :NBOX * 7].reshape(B, NBOX, 7)

    x48, conf = _tc_mlp(fsel.reshape(B * NBOX, D),
                        W1, b1, g1, be1, W2, b2, g2, be2, W3, b3, W4, b4)
    return (x48.reshape(B, NBOX, 7), conf, bsel)


# speedup vs baseline: 1.3791x; 1.3791x over previous
import jax
import jax.numpy as jnp
from jax import lax
from jax.experimental import pallas as pl
from jax.experimental.pallas import tpu as pltpu
from jax.experimental.pallas import tpu_sc as plsc

B, N, D = 8, 2048, 1024
NBOX = 6
THRESH = 0.5
L = 16
NCHUNK = N // L

_SUPP_OFF = 16.0
_PICKED = -1e30
_LIVE_MIN = -1.0


def _sc_nms_gather_body(scores_hbm, cols_hbm, feat_hbm,
                        fsel_hbm, bsel_hbm,
                        orig_v, s_v, col_v, x1_v, y1_v, x2_v, y2_v, area_v,
                        bsel_v, idx_v, frows_v, sem):
    nc = 2
    wid = lax.axis_index("s") * nc + lax.axis_index("c")
    lane = lax.iota(jnp.int32, L)

    @pl.when(wid < B)
    def _():
        b = wid
        pltpu.sync_copy(scores_hbm.at[b], orig_v)
        pltpu.sync_copy(scores_hbm.at[b], s_v)
        for j in range(7):
            pltpu.sync_copy(cols_hbm.at[j * B + b], col_v.at[pl.ds(j * N, N)])

        neg_inf = jnp.full((L,), -jnp.inf, jnp.float32)
        zeros_i = jnp.zeros((L,), jnp.int32)

        def bev_body(c, carry):
            bv, bi = carry
            o = c * L
            xc = col_v[pl.ds(o, L)]
            zc = col_v[pl.ds(2 * N + o, L)]
            ww = col_v[pl.ds(4 * N + o, L)]
            ll = col_v[pl.ds(5 * N + o, L)]
            x1 = xc - ll / 2.0
            y1 = zc - ww / 2.0
            x2 = xc + ll / 2.0
            y2 = zc + ww / 2.0
            sl = pl.ds(o, L)
            x1_v[sl] = x1
            y1_v[sl] = y1
            x2_v[sl] = x2
            y2_v[sl] = y2
            area_v[sl] = (x2 - x1) * (y2 - y1)
            v = s_v[sl]
            gt = v > bv
            return (jnp.where(gt, v, bv), jnp.where(gt, o + lane, bi))

        bv, bi = lax.fori_loop(0, NCHUNK, bev_body, (neg_inf, zeros_i))

        picks = zeros_i
        for r in range(NBOX):
            m = jnp.max(bv)
            i = jnp.min(jnp.where(bv == m, bi, N))
            picks = jnp.where(lane == r, i, picks)

            if r < NBOX - 1:
                iv = jnp.full((L,), i, jnp.int32)
                px1 = plsc.load_gather(x1_v, [iv])
                py1 = plsc.load_gather(y1_v, [iv])
                px2 = plsc.load_gather(x2_v, [iv])
                py2 = plsc.load_gather(y2_v, [iv])
                parea = plsc.load_gather(area_v, [iv])
                porig = plsc.load_gather(orig_v, [iv])
                plsc.store_scatter(s_v, [iv], jnp.full((L,), _PICKED, jnp.float32),
                                   mask=lane == 0)

                @pl.when(m > _LIVE_MIN)
                def _():
                    def supp_body(c, _):
                        sl = pl.ds(c * L, L)
                        jx1 = x1_v[sl]
                        jy1 = y1_v[sl]
                        jx2 = x2_v[sl]
                        jy2 = y2_v[sl]
                        jarea = area_v[sl]
                        jorig = orig_v[sl]
                        js = s_v[sl]
                        inter = (jnp.maximum(jnp.minimum(px2, jx2) - jnp.maximum(px1, jx1), 0.0)
                                 * jnp.maximum(jnp.minimum(py2, jy2) - jnp.maximum(py1, jy1), 0.0))
                        iou = inter / (parea + jarea - inter + 1e-8)
                        cond = (iou > THRESH) & (jorig < porig) & (js > _LIVE_MIN)
                        s_v[sl] = jnp.where(cond, js - _SUPP_OFF, js)
                        return 0

                    lax.fori_loop(0, NCHUNK, supp_body, 0)

                def amax_body(c, carry):
                    bv, bi = carry
                    v = s_v[pl.ds(c * L, L)]
                    gt = v > bv
                    return (jnp.where(gt, v, bv), jnp.where(gt, c * L + lane, bi))

                bv, bi = lax.fori_loop(0, NCHUNK, amax_body, (neg_inf, zeros_i))

        gidx = picks + b * N
        plsc.store_scatter(idx_v, [lane], gidx, mask=lane < 8)
        fcp = pltpu.async_copy(feat_hbm.at[idx_v], frows_v, sem)

        for c in range(7):
            vals = plsc.load_gather(col_v, [c * N + picks])
            plsc.store_scatter(bsel_v, [lane * 7 + c], vals, mask=lane < NBOX)
        pltpu.sync_copy(bsel_v, bsel_hbm.at[b])

        fcp.wait()
        pltpu.sync_copy(frows_v.at[pl.ds(0, NBOX)], fsel_hbm.at[b])


@jax.jit
def _sc_nms_gather(scores, cols, feat_flat):
    mesh = plsc.VectorSubcoreMesh(core_axis_name="c", subcore_axis_name="s")
    return pl.kernel(
        _sc_nms_gather_body,
        out_type=[
            jax.ShapeDtypeStruct((B, NBOX, D), jnp.float32),
            jax.ShapeDtypeStruct((B, 56), jnp.float32),
        ],
        mesh=mesh,
        compiler_params=pltpu.CompilerParams(needs_layout_passes=False),
        scratch_types=[
            pltpu.VMEM((N,), jnp.float32),
            pltpu.VMEM((N,), jnp.float32),
            pltpu.VMEM((7 * N,), jnp.float32),
            pltpu.VMEM((N,), jnp.float32),
            pltpu.VMEM((N,), jnp.float32),
            pltpu.VMEM((N,), jnp.float32),
            pltpu.VMEM((N,), jnp.float32),
            pltpu.VMEM((N,), jnp.float32),
            pltpu.VMEM((56,), jnp.float32),
            pltpu.VMEM((8,), jnp.int32),
            pltpu.VMEM((8, D), jnp.float32),
            pltpu.SemaphoreType.DMA,
        ],
    )(scores, cols, feat_flat)


def _tc_mlp_body(h_ref, w1_ref, b1_ref, g1_ref, be1_ref,
                 w2_ref, b2_ref, g2_ref, be2_ref,
                 w3_ref, b3_ref, w4_ref, b4_ref, x_ref, conf_ref):
    h = h_ref[...].reshape(B * NBOX, D)

    def bn(x, g, b):
        mu = jnp.mean(x, axis=0)
        var = jnp.mean((x - mu[None, :]) ** 2, axis=0)
        return (x - mu[None, :]) / jnp.sqrt(var + 1e-5) * g[None, :] + b[None, :]

    h1 = jnp.dot(h, w1_ref[...], preferred_element_type=jnp.float32) + b1_ref[...][None, :]
    h1 = jnp.maximum(bn(h1, g1_ref[...], be1_ref[...]), 0.0)
    h2 = jnp.dot(h1, w2_ref[...], preferred_element_type=jnp.float32) + b2_ref[...][None, :]
    h2 = jnp.maximum(bn(h2, g2_ref[...], be2_ref[...]), 0.0)
    x_ref[...] = (jnp.dot(h2, w3_ref[...], preferred_element_type=jnp.float32)
                  + b3_ref[...][None, :])
    logits = (jnp.dot(h2, w4_ref[...], preferred_element_type=jnp.float32)
              + b4_ref[...][None, :])
    mx = jnp.max(logits, axis=1, keepdims=True)
    e = jnp.exp(logits - mx)
    conf_ref[...] = e / jnp.sum(e, axis=1, keepdims=True)


@jax.jit
def _tc_mlp(h, W1, b1, g1, be1, W2, b2, g2, be2, W3, b3, W4, b4):
    return pl.pallas_call(
        _tc_mlp_body,
        out_shape=[
            jax.ShapeDtypeStruct((B * NBOX, 7), jnp.float32),
            jax.ShapeDtypeStruct((B * NBOX, 2), jnp.float32),
        ],
    )(h, W1, b1, g1, be1, W2, b2, g2, be2, W3, b3, W4, b4)


def kernel(features, boxes3d, scores, W1, b1, g1, be1, W2, b2, g2, be2, W3, b3, W4, b4):
    feat_flat = features.reshape(B * N, D)
    cols = jnp.moveaxis(boxes3d, 2, 0).reshape(7 * B, N)
    fsel, bsel_pad = _sc_nms_gather(scores, cols, feat_flat)
    bsel = bsel_pad[:, :NBOX * 7].reshape(B, NBOX, 7)

    x48, conf = _tc_mlp(fsel, W1, b1, g1, be1, W2, b2, g2, be2, W3, b3, W4, b4)
    return (x48.reshape(B, NBOX, 7), conf, bsel)


# speedup vs baseline: 1.4924x; 1.0822x over previous
import jax
import jax.numpy as jnp
from jax import lax
from jax.experimental import pallas as pl
from jax.experimental.pallas import tpu as pltpu
from jax.experimental.pallas import tpu_sc as plsc

B, N, D = 8, 2048, 1024
NBOX = 6
THRESH = 0.5
L = 16
NCHUNK = N // L

_SUPP_OFF = 16.0
_PICKED = -1e30
_LIVE_MIN = -1.0


def _sc_nms_gather_body(scores_hbm, cols_hbm, feat_hbm,
                        fsel_hbm, bsel_hbm,
                        orig_v, s_v, col_v, x1_v, y1_v, x2_v, y2_v, area_v,
                        bsel_v, idx_v, frows_v, sem):
    nc = 2
    wid = lax.axis_index("s") * nc + lax.axis_index("c")
    lane = lax.iota(jnp.int32, L)

    @pl.when(wid < B)
    def _():
        b = wid
        pltpu.sync_copy(scores_hbm.at[b], orig_v)
        pltpu.sync_copy(scores_hbm.at[b], s_v)
        pltpu.sync_copy(cols_hbm.at[b], col_v)

        neg_inf = jnp.full((L,), -jnp.inf, jnp.float32)
        zeros_i = jnp.zeros((L,), jnp.int32)

        def bev_body(c, carry):
            bv, bi = carry
            o = c * L
            xc = col_v[pl.ds(o, L)]
            zc = col_v[pl.ds(2 * N + o, L)]
            ww = col_v[pl.ds(4 * N + o, L)]
            ll = col_v[pl.ds(5 * N + o, L)]
            x1 = xc - ll / 2.0
            y1 = zc - ww / 2.0
            x2 = xc + ll / 2.0
            y2 = zc + ww / 2.0
            sl = pl.ds(o, L)
            x1_v[sl] = x1
            y1_v[sl] = y1
            x2_v[sl] = x2
            y2_v[sl] = y2
            area_v[sl] = (x2 - x1) * (y2 - y1)
            v = s_v[sl]
            gt = v > bv
            return (jnp.where(gt, v, bv), jnp.where(gt, o + lane, bi))

        bv, bi = lax.fori_loop(0, NCHUNK, bev_body, (neg_inf, zeros_i))

        picks = zeros_i
        for r in range(NBOX):
            m = jnp.max(bv)
            i = jnp.min(jnp.where(bv == m, bi, N))
            picks = jnp.where(lane == r, i, picks)

            if r < NBOX - 1:
                iv = jnp.full((L,), i, jnp.int32)
                px1 = plsc.load_gather(x1_v, [iv])
                py1 = plsc.load_gather(y1_v, [iv])
                px2 = plsc.load_gather(x2_v, [iv])
                py2 = plsc.load_gather(y2_v, [iv])
                parea = plsc.load_gather(area_v, [iv])
                porig = plsc.load_gather(orig_v, [iv])
                plsc.store_scatter(s_v, [iv], jnp.full((L,), _PICKED, jnp.float32),
                                   mask=lane == 0)

                @pl.when(m > _LIVE_MIN)
                def _():
                    def supp_body(c, _):
                        sl = pl.ds(c * L, L)
                        jx1 = x1_v[sl]
                        jy1 = y1_v[sl]
                        jx2 = x2_v[sl]
                        jy2 = y2_v[sl]
                        jarea = area_v[sl]
                        jorig = orig_v[sl]
                        js = s_v[sl]
                        inter = (jnp.maximum(jnp.minimum(px2, jx2) - jnp.maximum(px1, jx1), 0.0)
                                 * jnp.maximum(jnp.minimum(py2, jy2) - jnp.maximum(py1, jy1), 0.0))
                        iou = inter / (parea + jarea - inter + 1e-8)
                        cond = (iou > THRESH) & (jorig < porig) & (js > _LIVE_MIN)
                        s_v[sl] = jnp.where(cond, js - _SUPP_OFF, js)
                        return 0

                    lax.fori_loop(0, NCHUNK, supp_body, 0)

                def amax_body(c, carry):
                    bv, bi = carry
                    v = s_v[pl.ds(c * L, L)]
                    gt = v > bv
                    return (jnp.where(gt, v, bv), jnp.where(gt, c * L + lane, bi))

                bv, bi = lax.fori_loop(0, NCHUNK, amax_body, (neg_inf, zeros_i))

        gidx = picks + b * N
        plsc.store_scatter(idx_v, [lane], gidx, mask=lane < 8)
        fcp = pltpu.async_copy(feat_hbm.at[idx_v], frows_v, sem)

        for c in range(7):
            vals = plsc.load_gather(col_v, [c * N + picks])
            plsc.store_scatter(bsel_v, [lane * 7 + c], vals, mask=lane < NBOX)
        pltpu.sync_copy(bsel_v, bsel_hbm.at[b])

        fcp.wait()
        pltpu.sync_copy(frows_v.at[pl.ds(0, NBOX)], fsel_hbm.at[b])


@jax.jit
def _sc_nms_gather(scores, cols, feat_flat):
    mesh = plsc.VectorSubcoreMesh(core_axis_name="c", subcore_axis_name="s")
    return pl.kernel(
        _sc_nms_gather_body,
        out_type=[
            jax.ShapeDtypeStruct((B, NBOX, D), jnp.float32),
            jax.ShapeDtypeStruct((B, 56), jnp.float32),
        ],
        mesh=mesh,
        compiler_params=pltpu.CompilerParams(needs_layout_passes=False),
        scratch_types=[
            pltpu.VMEM((N,), jnp.float32),
            pltpu.VMEM((N,), jnp.float32),
            pltpu.VMEM((7 * N,), jnp.float32),
            pltpu.VMEM((N,), jnp.float32),
            pltpu.VMEM((N,), jnp.float32),
            pltpu.VMEM((N,), jnp.float32),
            pltpu.VMEM((N,), jnp.float32),
            pltpu.VMEM((N,), jnp.float32),
            pltpu.VMEM((56,), jnp.float32),
            pltpu.VMEM((8,), jnp.int32),
            pltpu.VMEM((8, D), jnp.float32),
            pltpu.SemaphoreType.DMA,
        ],
    )(scores, cols, feat_flat)


def _tc_mlp_body(h_ref, w1_ref, b1_ref, g1_ref, be1_ref,
                 w2_ref, b2_ref, g2_ref, be2_ref,
                 w3_ref, b3_ref, w4_ref, b4_ref, x_ref, conf_ref):
    h = h_ref[...].reshape(B * NBOX, D)

    def bn(x, g, b):
        mu = jnp.mean(x, axis=0)
        var = jnp.mean((x - mu[None, :]) ** 2, axis=0)
        return (x - mu[None, :]) / jnp.sqrt(var + 1e-5) * g[None, :] + b[None, :]

    h1 = jnp.dot(h, w1_ref[...], preferred_element_type=jnp.float32) + b1_ref[...][None, :]
    h1 = jnp.maximum(bn(h1, g1_ref[...], be1_ref[...]), 0.0)
    h2 = jnp.dot(h1, w2_ref[...], preferred_element_type=jnp.float32) + b2_ref[...][None, :]
    h2 = jnp.maximum(bn(h2, g2_ref[...], be2_ref[...]), 0.0)
    x_ref[...] = (jnp.dot(h2, w3_ref[...], preferred_element_type=jnp.float32)
                  + b3_ref[...][None, :])
    logits = (jnp.dot(h2, w4_ref[...], preferred_element_type=jnp.float32)
              + b4_ref[...][None, :])
    mx = jnp.max(logits, axis=1, keepdims=True)
    e = jnp.exp(logits - mx)
    conf_ref[...] = e / jnp.sum(e, axis=1, keepdims=True)


@jax.jit
def _tc_mlp(h, W1, b1, g1, be1, W2, b2, g2, be2, W3, b3, W4, b4):
    return pl.pallas_call(
        _tc_mlp_body,
        out_shape=[
            jax.ShapeDtypeStruct((B * NBOX, 7), jnp.float32),
            jax.ShapeDtypeStruct((B * NBOX, 2), jnp.float32),
        ],
    )(h, W1, b1, g1, be1, W2, b2, g2, be2, W3, b3, W4, b4)


def kernel(features, boxes3d, scores, W1, b1, g1, be1, W2, b2, g2, be2, W3, b3, W4, b4):
    feat_flat = features.reshape(B * N, D)
    cols = jnp.moveaxis(boxes3d, 2, 1).reshape(B, 7 * N)
    fsel, bsel_pad = _sc_nms_gather(scores, cols, feat_flat)
    bsel = bsel_pad[:, :NBOX * 7].reshape(B, NBOX, 7)

    x48, conf = _tc_mlp(fsel, W1, b1, g1, be1, W2, b2, g2, be2, W3, b3, W4, b4)
    return (x48.reshape(B, NBOX, 7), conf, bsel)


# speedup vs baseline: 1.5412x; 1.0327x over previous
import jax
import jax.numpy as jnp
from jax import lax
from jax.experimental import pallas as pl
from jax.experimental.pallas import tpu as pltpu
from jax.experimental.pallas import tpu_sc as plsc

B, N, D = 8, 2048, 1024
NBOX = 6
THRESH = 0.5
L = 16
NCHUNK = N // L

_SUPP_OFF = 16.0
_PICKED = -1e30
_LIVE_MIN = -1.0


def _sc_nms_gather_body(scores_hbm, cols_hbm, feat_hbm,
                        fsel_hbm, bsel_hbm,
                        s_v, col_v, x1_v, y1_v, x2_v, y2_v, area_v,
                        bsel_v, idx_v, frows_v, sem):
    nc = 2
    wid = lax.axis_index("s") * nc + lax.axis_index("c")
    lane = lax.iota(jnp.int32, L)

    @pl.when(wid < B)
    def _():
        b = wid
        pltpu.sync_copy(scores_hbm.at[b], s_v)
        pltpu.sync_copy(cols_hbm.at[b], col_v)

        neg_inf = jnp.full((L,), -jnp.inf, jnp.float32)
        zeros_i = jnp.zeros((L,), jnp.int32)

        def bev_body(c, carry):
            bv, bi = carry
            o = c * L
            xc = col_v[pl.ds(o, L)]
            zc = col_v[pl.ds(2 * N + o, L)]
            ww = col_v[pl.ds(4 * N + o, L)]
            ll = col_v[pl.ds(5 * N + o, L)]
            x1 = xc - ll / 2.0
            y1 = zc - ww / 2.0
            x2 = xc + ll / 2.0
            y2 = zc + ww / 2.0
            sl = pl.ds(o, L)
            x1_v[sl] = x1
            y1_v[sl] = y1
            x2_v[sl] = x2
            y2_v[sl] = y2
            area_v[sl] = (x2 - x1) * (y2 - y1)
            v = s_v[sl]
            gt = v > bv
            return (jnp.where(gt, v, bv), jnp.where(gt, o + lane, bi))

        bv, bi = lax.fori_loop(0, NCHUNK, bev_body, (neg_inf, zeros_i))

        picks = zeros_i
        for r in range(NBOX):
            m = jnp.max(bv)
            i = jnp.min(jnp.where(bv == m, bi, N))
            picks = jnp.where(lane == r, i, picks)

            if r < NBOX - 1:
                iv = jnp.full((L,), i, jnp.int32)
                px1 = plsc.load_gather(x1_v, [iv])
                py1 = plsc.load_gather(y1_v, [iv])
                px2 = plsc.load_gather(x2_v, [iv])
                py2 = plsc.load_gather(y2_v, [iv])
                parea = plsc.load_gather(area_v, [iv])
                plsc.store_scatter(s_v, [iv], jnp.full((L,), _PICKED, jnp.float32),
                                   mask=lane == 0)

                @pl.when(m > _LIVE_MIN)
                def _():
                    mv = jnp.full((L,), m, jnp.float32)

                    def supp_body(c, _):
                        sl = pl.ds(c * L, L)
                        jx1 = x1_v[sl]
                        jy1 = y1_v[sl]
                        jx2 = x2_v[sl]
                        jy2 = y2_v[sl]
                        jarea = area_v[sl]
                        js = s_v[sl]
                        inter = (jnp.maximum(jnp.minimum(px2, jx2) - jnp.maximum(px1, jx1), 0.0)
                                 * jnp.maximum(jnp.minimum(py2, jy2) - jnp.maximum(py1, jy1), 0.0))
                        iou = inter / (parea + jarea - inter + 1e-8)
                        cond = (iou > THRESH) & (js < mv) & (js > _LIVE_MIN)
                        s_v[sl] = jnp.where(cond, js - _SUPP_OFF, js)
                        return 0

                    lax.fori_loop(0, NCHUNK, supp_body, 0)

                def amax_body(c, carry):
                    bv, bi = carry
                    v = s_v[pl.ds(c * L, L)]
                    gt = v > bv
                    return (jnp.where(gt, v, bv), jnp.where(gt, c * L + lane, bi))

                bv, bi = lax.fori_loop(0, NCHUNK, amax_body, (neg_inf, zeros_i))

        gidx = picks + b * N
        plsc.store_scatter(idx_v, [lane], gidx, mask=lane < 8)
        fcp = pltpu.async_copy(feat_hbm.at[idx_v], frows_v, sem)

        for c in range(7):
            vals = plsc.load_gather(col_v, [c * N + picks])
            plsc.store_scatter(bsel_v, [lane * 7 + c], vals, mask=lane < NBOX)
        pltpu.sync_copy(bsel_v, bsel_hbm.at[b])

        fcp.wait()
        pltpu.sync_copy(frows_v.at[pl.ds(0, NBOX)], fsel_hbm.at[b])


@jax.jit
def _sc_nms_gather(scores, cols, feat_flat):
    mesh = plsc.VectorSubcoreMesh(core_axis_name="c", subcore_axis_name="s")
    return pl.kernel(
        _sc_nms_gather_body,
        out_type=[
            jax.ShapeDtypeStruct((B, NBOX, D), jnp.float32),
            jax.ShapeDtypeStruct((B, 56), jnp.float32),
        ],
        mesh=mesh,
        compiler_params=pltpu.CompilerParams(needs_layout_passes=False),
        scratch_types=[
            pltpu.VMEM((N,), jnp.float32),
            pltpu.VMEM((7 * N,), jnp.float32),
            pltpu.VMEM((N,), jnp.float32),
            pltpu.VMEM((N,), jnp.float32),
            pltpu.VMEM((N,), jnp.float32),
            pltpu.VMEM((N,), jnp.float32),
            pltpu.VMEM((N,), jnp.float32),
            pltpu.VMEM((56,), jnp.float32),
            pltpu.VMEM((8,), jnp.int32),
            pltpu.VMEM((8, D), jnp.float32),
            pltpu.SemaphoreType.DMA,
        ],
    )(scores, cols, feat_flat)


def _tc_mlp_body(h_ref, w1_ref, b1_ref, g1_ref, be1_ref,
                 w2_ref, b2_ref, g2_ref, be2_ref,
                 w3_ref, b3_ref, w4_ref, b4_ref, x_ref, conf_ref):
    h = h_ref[...].reshape(B * NBOX, D)

    def bn(x, g, b):
        mu = jnp.mean(x, axis=0)
        var = jnp.mean((x - mu[None, :]) ** 2, axis=0)
        return (x - mu[None, :]) / jnp.sqrt(var + 1e-5) * g[None, :] + b[None, :]

    h1 = jnp.dot(h, w1_ref[...], preferred_element_type=jnp.float32) + b1_ref[...][None, :]
    h1 = jnp.maximum(bn(h1, g1_ref[...], be1_ref[...]), 0.0)
    h2 = jnp.dot(h1, w2_ref[...], preferred_element_type=jnp.float32) + b2_ref[...][None, :]
    h2 = jnp.maximum(bn(h2, g2_ref[...], be2_ref[...]), 0.0)
    x_ref[...] = (jnp.dot(h2, w3_ref[...], preferred_element_type=jnp.float32)
                  + b3_ref[...][None, :])
    logits = (jnp.dot(h2, w4_ref[...], preferred_element_type=jnp.float32)
              + b4_ref[...][None, :])
    mx = jnp.max(logits, axis=1, keepdims=True)
    e = jnp.exp(logits - mx)
    conf_ref[...] = e / jnp.sum(e, axis=1, keepdims=True)


@jax.jit
def _tc_mlp(h, W1, b1, g1, be1, W2, b2, g2, be2, W3, b3, W4, b4):
    return pl.pallas_call(
        _tc_mlp_body,
        out_shape=[
            jax.ShapeDtypeStruct((B * NBOX, 7), jnp.float32),
            jax.ShapeDtypeStruct((B * NBOX, 2), jnp.float32),
        ],
    )(h, W1, b1, g1, be1, W2, b2, g2, be2, W3, b3, W4, b4)


def kernel(features, boxes3d, scores, W1, b1, g1, be1, W2, b2, g2, be2, W3, b3, W4, b4):
    feat_flat = features.reshape(B * N, D)
    cols = jnp.moveaxis(boxes3d, 2, 1).reshape(B, 7 * N)
    fsel, bsel_pad = _sc_nms_gather(scores, cols, feat_flat)
    bsel = bsel_pad[:, :NBOX * 7].reshape(B, NBOX, 7)

    x48, conf = _tc_mlp(fsel, W1, b1, g1, be1, W2, b2, g2, be2, W3, b3, W4, b4)
    return (x48.reshape(B, NBOX, 7), conf, bsel)
